# baseline (speedup 1.0000x reference)
import functools

import jax
import jax.numpy as jnp
from jax import lax
from jax.experimental import pallas as pl
from jax.experimental.pallas import tpu as pltpu
from jax.experimental.pallas import tpu_sc as plsc

NC = 2
NS = 16
NW = NC * NS
CHUNK = 64
NBUF = 10
K = NBUF - 1


def kernel(x, weight):
    BATCH, H = x.shape
    B = x.size
    D = weight.shape[1]
    n_chunks = B // (NW * CHUNK)
    assert B == NW * n_chunks * CHUNK
    assert (n_chunks - NBUF) % NBUF == 0 and n_chunks > 2 * NBUF
    idx = x.T.reshape(NW, n_chunks, CHUNK).astype(jnp.int32)

    mesh = plsc.VectorSubcoreMesh(
        core_axis_name="c", subcore_axis_name="s",
        num_cores=NC, num_subcores=NS,
    )

    @functools.partial(
        pl.kernel,
        out_type=jax.ShapeDtypeStruct((B, D), jnp.float32),
        mesh=mesh,
        scratch_types=[
            pltpu.VMEM((n_chunks, CHUNK), jnp.int32),
            pltpu.VMEM((NBUF, CHUNK, D), jnp.float32),
            pltpu.SemaphoreType.DMA((NBUF,)),
            pltpu.SemaphoreType.DMA((NBUF,)),
        ],
    )
    def emb(x_hbm, w_hbm, out_hbm, idx_v, rows_v, gsem, ssem):
        wid = lax.axis_index("s") * NC + lax.axis_index("c")
        base = wid * (n_chunks * CHUNK)
        pltpu.sync_copy(x_hbm.at[wid], idx_v)

        def start_gather(j, b):
            pltpu.async_copy(w_hbm.at[idx_v.at[j]], rows_v.at[b], gsem.at[b])

        def wait_gather(b):
            pltpu.make_async_copy(
                w_hbm.at[pl.ds(0, CHUNK)], rows_v.at[b], gsem.at[b]
            ).wait()

        def start_store(j, b):
            pltpu.async_copy(
                rows_v.at[b], out_hbm.at[pl.ds(base + j * CHUNK, CHUNK)],
                ssem.at[b],
            )

        def wait_store(b):
            pltpu.make_async_copy(
                w_hbm.at[pl.ds(0, CHUNK)], rows_v.at[b], ssem.at[b]
            ).wait()

        for j in range(K):
            start_gather(j, j)
        start_gather(K, K)
        wait_gather(0)
        start_store(0, 0)

        @pl.loop(NBUF, n_chunks, step=NBUF)
        def group(j0):
            for b in range(NBUF):
                j = j0 + b
                wait_store(b)
                start_gather(j, b)
                bc = (b + 1) % NBUF
                wait_gather(bc)
                start_store(j - K, bc)

        for j in range(n_chunks, n_chunks + K):
            bc = (j - K) % NBUF
            wait_gather(bc)
            start_store(j - K, bc)
        for j in range(n_chunks - NBUF, n_chunks):
            wait_store(j % NBUF)

    out = emb(idx, weight)
    return out.reshape(H, BATCH, D).transpose(1, 0, 2)
